# baseline (speedup 1.0000x reference)
import functools

import jax
import jax.numpy as jnp
from jax import lax
from jax.experimental import pallas as pl
from jax.experimental.pallas import tpu as pltpu
from jax.experimental.pallas import tpu_sc as plsc

_N = 10000
_E = 320000
_D = 128
_G = 64
_NP = 10240
_NC = 2
_NS = 16
_NW = _NC * _NS
_EPW = _E // _NW
_C = 80
_CPW = _EPW // _C
_NSEG = 5
_CSEG = _CPW // _NSEG
_RPT = _NP // _NS


def _fill1d(ref, n, value):
    v = jnp.full((16,), value, jnp.float32)

    def body(i, carry):
        ref[pl.ds(i * 16, 16)] = v
        return carry

    lax.fori_loop(0, n // 16, body, None)


def _sc_mesh():
    return plsc.VectorSubcoreMesh(
        core_axis_name="c", subcore_axis_name="s",
        num_cores=_NC, num_subcores=_NS)


def _deg(dstw):
    @functools.partial(
        pl.kernel,
        out_type=jax.ShapeDtypeStruct((_NC * _NP,), jnp.float32),
        mesh=_sc_mesh(),
        scratch_types=[
            pltpu.VMEM((_CSEG, _C), jnp.int32),
            pltpu.VMEM((_C,), jnp.float32),
            pltpu.VMEM((_RPT,), jnp.float32),
            pltpu.VMEM_SHARED((_NP,), jnp.float32),
            pltpu.SemaphoreType.DMA,
        ],
    )
    def run(dst_hbm, deg_out, didx, ones_v, init_v, acc_sp, sem):
        cid = lax.axis_index("c")
        sid = lax.axis_index("s")
        w = cid * _NS + sid
        _fill1d(ones_v, _C, 1.0)
        _fill1d(init_v, _RPT, 0.5)
        pltpu.sync_copy(init_v, acc_sp.at[pl.ds(sid * _RPT, _RPT)])
        plsc.subcore_barrier()

        def seg(si, carry):
            pltpu.sync_copy(dst_hbm.at[w, si], didx)

            def group(k, c2):
                descs = [
                    pltpu.async_copy(ones_v, acc_sp.at[didx.at[4 * k + j]],
                                     sem, add=True)
                    for j in range(4)
                ]
                for d in descs:
                    d.wait()
                return c2

            lax.fori_loop(0, _CSEG // 4, group, None)
            pltpu.sync_copy(ones_v, acc_sp.at[didx.at[_CSEG - 1]], add=True)
            return carry

        lax.fori_loop(0, _NSEG, seg, None)
        plsc.subcore_barrier()
        pltpu.sync_copy(acc_sp.at[pl.ds(sid * _RPT, _RPT)],
                        deg_out.at[pl.ds(cid * _NP + sid * _RPT, _RPT)])

    return run(dstw)


def _prop(hp, srcw, dstw):
    @functools.partial(
        pl.kernel,
        out_type=jax.ShapeDtypeStruct((_NC, _NP, _D), jnp.float32),
        mesh=_sc_mesh(),
        scratch_types=[
            pltpu.VMEM((_CSEG, _C), jnp.int32),
            pltpu.VMEM((_CSEG, _C), jnp.int32),
            pltpu.VMEM((_C, _D), jnp.float32),
            pltpu.VMEM((_C, _D), jnp.float32),
            pltpu.VMEM((_C, _D), jnp.float32),
            pltpu.VMEM_SHARED((_NP, _D), jnp.float32),
            [pltpu.SemaphoreType.DMA] * 3,
            [pltpu.SemaphoreType.DMA] * 3,
        ],
    )
    def run(hp_hbm, src_hbm, dst_hbm, acc_out, sidx, didx, r0, r1, r2,
            acc_sp, gsems, ssems):
        cid = lax.axis_index("c")
        sid = lax.axis_index("s")
        w = cid * _NS + sid
        row0 = sid * _RPT
        rows = (r0, r1, r2)

        @pl.when(cid == 0)
        def _():
            pltpu.sync_copy(hp_hbm.at[pl.ds(row0, _RPT)],
                            acc_sp.at[pl.ds(row0, _RPT)])

        @pl.when(cid == 1)
        def _():
            def zrow(i, carry):
                r0[i, pl.ds(0, 16)] = jnp.zeros((16,), jnp.float32)
                for jj in range(1, _D // 16):
                    r0[i, pl.ds(jj * 16, 16)] = jnp.zeros((16,), jnp.float32)
                return carry

            lax.fori_loop(0, _C, zrow, None)
            for j in range(_RPT // _C):
                pltpu.sync_copy(r0, acc_sp.at[pl.ds(row0 + j * _C, _C)])

        plsc.subcore_barrier()

        def g(c, b):
            pltpu.async_copy(hp_hbm.at[sidx.at[c]], rows[b], gsems[b])

        def wg(c, b):
            pltpu.make_async_copy(
                hp_hbm.at[sidx.at[c]], rows[b], gsems[b]).wait()

        def sc_(c, b):
            pltpu.async_copy(rows[b], acc_sp.at[didx.at[c]], ssems[b],
                             add=True)

        def ws(c, b):
            pltpu.make_async_copy(
                rows[b], acc_sp.at[didx.at[c]], ssems[b]).wait()

        def seg(si, carry):
            pltpu.sync_copy(src_hbm.at[w, si], sidx)
            pltpu.sync_copy(dst_hbm.at[w, si], didx)
            g(0, 0)
            g(1, 1)
            wg(0, 0)
            sc_(0, 0)
            g(2, 2)

            def trip(k, c3):
                c = 3 * k + 1
                for j, b in ((0, 1), (1, 2), (2, 0)):
                    wg(c + j, b)
                    sc_(c + j, b)
                    bp = (b + 2) % 3
                    ws(c + j - 1, bp)
                    g(c + j + 2, bp)
                return c3

            lax.fori_loop(0, (_CSEG - 4) // 3, trip, None)
            wg(_CSEG - 3, 1)
            sc_(_CSEG - 3, 1)
            ws(_CSEG - 4, 0)
            g(_CSEG - 1, 0)
            wg(_CSEG - 2, 2)
            sc_(_CSEG - 2, 2)
            wg(_CSEG - 1, 0)
            sc_(_CSEG - 1, 0)
            ws(_CSEG - 3, 1)
            ws(_CSEG - 2, 2)
            ws(_CSEG - 1, 0)
            return carry

        lax.fori_loop(0, _NSEG, seg, None)
        plsc.subcore_barrier()
        pltpu.sync_copy(acc_sp.at[pl.ds(row0, _RPT)],
                        acc_out.at[cid, pl.ds(row0, _RPT)])

    return run(hp, srcw, dstw)


def _dinv_of(deg_ref):
    d = deg_ref[0] + deg_ref[1]
    return lax.rsqrt(d)


def _mm_scale_body(deg_ref, x_ref, w_ref, o_ref):
    h = jnp.dot(x_ref[...], w_ref[...],
                preferred_element_type=jnp.float32,
                precision=lax.Precision.DEFAULT)
    o_ref[...] = h * _dinv_of(deg_ref)


def _layer2_body(deg_ref, acc_ref, w_ref, b_ref, o_ref):
    dinv = _dinv_of(deg_ref)
    a = acc_ref[0] + acc_ref[1]
    t = jnp.maximum(a * dinv + b_ref[...], 0.0)
    o_ref[...] = jnp.dot(t, w_ref[...],
                         preferred_element_type=jnp.float32,
                         precision=lax.Precision.DEFAULT) * dinv


def _tc(body, out_shape, *args):
    return pl.pallas_call(
        body, out_shape=jax.ShapeDtypeStruct(out_shape, jnp.float32))(*args)


_NBLK = 8
_BR = _NP // _NBLK


def _mm_scale_grid(deg, xp, W1):
    return pl.pallas_call(
        _mm_scale_body,
        grid=(_NBLK,),
        in_specs=[
            pl.BlockSpec((_NC, _BR, 1), lambda i: (0, i, 0)),
            pl.BlockSpec((_BR, _D), lambda i: (i, 0)),
            pl.BlockSpec((_D, _D), lambda i: (0, 0)),
        ],
        out_specs=pl.BlockSpec((_BR, _D), lambda i: (i, 0)),
        out_shape=jax.ShapeDtypeStruct((_NP, _D), jnp.float32),
    )(deg, xp, W1)


def _layer2_grid(deg, acc, W2, b1r):
    return pl.pallas_call(
        _layer2_body,
        grid=(_NBLK,),
        in_specs=[
            pl.BlockSpec((_NC, _BR, 1), lambda i: (0, i, 0)),
            pl.BlockSpec((_NC, _BR, _D), lambda i: (0, i, 0)),
            pl.BlockSpec((_D, _D), lambda i: (0, 0)),
            pl.BlockSpec((1, _D), lambda i: (0, 0)),
        ],
        out_specs=pl.BlockSpec((_BR, _D), lambda i: (i, 0)),
        out_shape=jax.ShapeDtypeStruct((_NP, _D), jnp.float32),
    )(deg, acc, W2, b1r)


def _final_grid_body(deg_ref, acc_ref, b_ref, batch_ref, o_ref, pooled_s,
                     counts_s):
    i = pl.program_id(0)
    dinv = _dinv_of(deg_ref)
    a = acc_ref[0] + acc_ref[1]
    h = a * dinv + b_ref[...]
    gids = lax.broadcasted_iota(jnp.int32, (1, _G), 1)
    onehot = (batch_ref[...] == gids).astype(jnp.float32)
    dn = (((0,), (0,)), ((), ()))
    pooled = lax.dot_general(onehot, h, dn,
                             preferred_element_type=jnp.float32,
                             precision=lax.Precision.DEFAULT)
    ones = jnp.ones((_BR, 1), jnp.float32)
    counts = lax.dot_general(onehot, ones, dn,
                             preferred_element_type=jnp.float32,
                             precision=lax.Precision.DEFAULT)

    @pl.when(i == 0)
    def _():
        pooled_s[...] = jnp.zeros_like(pooled_s)
        counts_s[...] = jnp.zeros_like(counts_s)

    pooled_s[...] += pooled
    counts_s[...] += counts

    @pl.when(i == _NBLK - 1)
    def _():
        o_ref[...] = pooled_s[...] / jnp.maximum(counts_s[...], 1.0)


def _final_grid(deg, acc, b2r, batch2d):
    return pl.pallas_call(
        _final_grid_body,
        grid=(_NBLK,),
        in_specs=[
            pl.BlockSpec((_NC, _BR, 1), lambda i: (0, i, 0)),
            pl.BlockSpec((_NC, _BR, _D), lambda i: (0, i, 0)),
            pl.BlockSpec((1, _D), lambda i: (0, 0)),
            pl.BlockSpec((_BR, 1), lambda i: (i, 0)),
        ],
        out_specs=pl.BlockSpec((_G, _D), lambda i: (0, 0)),
        out_shape=jax.ShapeDtypeStruct((_G, _D), jnp.float32),
        scratch_shapes=[
            pltpu.VMEM((_G, _D), jnp.float32),
            pltpu.VMEM((_G, 1), jnp.float32),
        ],
    )(deg, acc, b2r, batch2d)


def kernel(x, edge_index, batch, W1, b1, W2, b2):
    srcw = edge_index[0].reshape(_NW, _NSEG, _CSEG, _C)
    dstw = edge_index[1].reshape(_NW, _NSEG, _CSEG, _C)
    xp = jnp.concatenate([x, jnp.zeros((_NP - _N, _D), x.dtype)])
    batch2d = jnp.concatenate(
        [batch, jnp.full((_NP - _N,), _G, batch.dtype)]).reshape(_NP, 1)
    b1r = b1.reshape(1, _D)
    b2r = b2.reshape(1, _D)

    deg = _deg(dstw).reshape(_NC, _NP, 1)
    hp1 = _mm_scale_grid(deg, xp, W1)
    acc1 = _prop(hp1, srcw, dstw)
    hp2 = _layer2_grid(deg, acc1, W2, b1r)
    acc2 = _prop(hp2, srcw, dstw)
    return _final_grid(deg, acc2, b2r, batch2d)


# speedup vs baseline: 1.0036x; 1.0036x over previous
import functools

import jax
import jax.numpy as jnp
from jax import lax
from jax.experimental import pallas as pl
from jax.experimental.pallas import tpu as pltpu
from jax.experimental.pallas import tpu_sc as plsc

_N = 10000
_E = 320000
_D = 128
_G = 64
_NP = 10240
_NC = 2
_NS = 16
_NW = _NC * _NS
_EPW = _E // _NW
_C = 80
_CPW = _EPW // _C
_NSEG = 5
_CSEG = _CPW // _NSEG
_RPT = _NP // _NS


def _fill1d(ref, n, value):
    v = jnp.full((16,), value, jnp.float32)

    def body(i, carry):
        ref[pl.ds(i * 16, 16)] = v
        return carry

    lax.fori_loop(0, n // 16, body, None)


def _sc_mesh():
    return plsc.VectorSubcoreMesh(
        core_axis_name="c", subcore_axis_name="s",
        num_cores=_NC, num_subcores=_NS)


def _deg(dstw):
    @functools.partial(
        pl.kernel,
        out_type=jax.ShapeDtypeStruct((_NC * _NP,), jnp.float32),
        mesh=_sc_mesh(),
        scratch_types=[
            pltpu.VMEM((_CSEG, _C), jnp.int32),
            pltpu.VMEM((_C,), jnp.float32),
            pltpu.VMEM((_RPT,), jnp.float32),
            pltpu.VMEM_SHARED((_NP,), jnp.float32),
            pltpu.SemaphoreType.DMA,
        ],
    )
    def run(dst_hbm, deg_out, didx, ones_v, init_v, acc_sp, sem):
        cid = lax.axis_index("c")
        sid = lax.axis_index("s")
        w = cid * _NS + sid
        _fill1d(ones_v, _C, 1.0)
        _fill1d(init_v, _RPT, 0.5)
        pltpu.sync_copy(init_v, acc_sp.at[pl.ds(sid * _RPT, _RPT)])
        plsc.subcore_barrier()

        def seg(si, carry):
            pltpu.sync_copy(dst_hbm.at[w, si], didx)

            def group(k, c2):
                descs = [
                    pltpu.async_copy(ones_v, acc_sp.at[didx.at[4 * k + j]],
                                     sem, add=True)
                    for j in range(4)
                ]
                for d in descs:
                    d.wait()
                return c2

            lax.fori_loop(0, _CSEG // 4, group, None)
            pltpu.sync_copy(ones_v, acc_sp.at[didx.at[_CSEG - 1]], add=True)
            return carry

        lax.fori_loop(0, _NSEG, seg, None)
        plsc.subcore_barrier()
        pltpu.sync_copy(acc_sp.at[pl.ds(sid * _RPT, _RPT)],
                        deg_out.at[pl.ds(cid * _NP + sid * _RPT, _RPT)])

    return run(dstw)


def _prop(hp, srcw, dstw):
    @functools.partial(
        pl.kernel,
        out_type=jax.ShapeDtypeStruct((_NC, _NP, _D), jnp.float32),
        mesh=_sc_mesh(),
        scratch_types=[
            pltpu.VMEM((_CSEG, _C), jnp.int32),
            pltpu.VMEM((_CSEG, _C), jnp.int32),
            pltpu.VMEM((_C, _D), jnp.float32),
            pltpu.VMEM((_C, _D), jnp.float32),
            pltpu.VMEM((_C, _D), jnp.float32),
            pltpu.VMEM_SHARED((_NP, _D), jnp.float32),
            [pltpu.SemaphoreType.DMA] * 3,
            [pltpu.SemaphoreType.DMA] * 3,
        ],
    )
    def run(hp_hbm, src_hbm, dst_hbm, acc_out, sidx, didx, r0, r1, r2,
            acc_sp, gsems, ssems):
        cid = lax.axis_index("c")
        sid = lax.axis_index("s")
        w = cid * _NS + sid
        row0 = sid * _RPT
        rows = (r0, r1, r2)

        @pl.when(cid == 0)
        def _():
            pltpu.sync_copy(hp_hbm.at[pl.ds(row0, _RPT)],
                            acc_sp.at[pl.ds(row0, _RPT)])

        @pl.when(cid == 1)
        def _():
            def zrow(i, carry):
                r0[i, pl.ds(0, 16)] = jnp.zeros((16,), jnp.float32)
                for jj in range(1, _D // 16):
                    r0[i, pl.ds(jj * 16, 16)] = jnp.zeros((16,), jnp.float32)
                return carry

            lax.fori_loop(0, _C, zrow, None)
            for j in range(_RPT // _C):
                pltpu.sync_copy(r0, acc_sp.at[pl.ds(row0 + j * _C, _C)])

        plsc.subcore_barrier()

        def g(c, b):
            pltpu.async_copy(hp_hbm.at[sidx.at[c]], rows[b], gsems[b])

        def wg(c, b):
            pltpu.make_async_copy(
                hp_hbm.at[sidx.at[c]], rows[b], gsems[b]).wait()

        def sc_(c, b):
            pltpu.async_copy(rows[b], acc_sp.at[didx.at[c]], ssems[b],
                             add=True)

        def ws(c, b):
            pltpu.make_async_copy(
                rows[b], acc_sp.at[didx.at[c]], ssems[b]).wait()

        def seg(si, carry):
            pltpu.sync_copy(src_hbm.at[w, si], sidx)
            pltpu.sync_copy(dst_hbm.at[w, si], didx)
            g(0, 0)
            g(1, 1)
            wg(0, 0)
            sc_(0, 0)
            g(2, 2)

            def trip(k, c3):
                c = 3 * k + 1
                for j, b in ((0, 1), (1, 2), (2, 0)):
                    wg(c + j, b)
                    sc_(c + j, b)
                    bp = (b + 2) % 3
                    ws(c + j - 1, bp)
                    g(c + j + 2, bp)
                return c3

            lax.fori_loop(0, (_CSEG - 4) // 3, trip, None)
            wg(_CSEG - 3, 1)
            sc_(_CSEG - 3, 1)
            ws(_CSEG - 4, 0)
            g(_CSEG - 1, 0)
            wg(_CSEG - 2, 2)
            sc_(_CSEG - 2, 2)
            wg(_CSEG - 1, 0)
            sc_(_CSEG - 1, 0)
            ws(_CSEG - 3, 1)
            ws(_CSEG - 2, 2)
            ws(_CSEG - 1, 0)
            return carry

        lax.fori_loop(0, _NSEG, seg, None)
        plsc.subcore_barrier()
        pltpu.sync_copy(acc_sp.at[pl.ds(row0, _RPT)],
                        acc_out.at[cid, pl.ds(row0, _RPT)])

    return run(hp, srcw, dstw)


def _dinv_of(deg_ref):
    d = deg_ref[0] + deg_ref[1]
    return lax.rsqrt(d)


def _mm_scale_body(deg_ref, x_ref, w_ref, o_ref):
    h = jnp.dot(x_ref[...], w_ref[...],
                preferred_element_type=jnp.float32,
                precision=lax.Precision.DEFAULT)
    o_ref[...] = h * _dinv_of(deg_ref)


def _layer2_body(deg_ref, acc_ref, w_ref, b_ref, o_ref):
    dinv = _dinv_of(deg_ref)
    a = acc_ref[0] + acc_ref[1]
    t = jnp.maximum(a * dinv + b_ref[...], 0.0)
    o_ref[...] = jnp.dot(t, w_ref[...],
                         preferred_element_type=jnp.float32,
                         precision=lax.Precision.DEFAULT) * dinv


def _tc(body, out_shape, *args):
    return pl.pallas_call(
        body, out_shape=jax.ShapeDtypeStruct(out_shape, jnp.float32))(*args)


def _final_body(deg_ref, acc_ref, b_ref, batch_ref, o_ref):
    dinv = _dinv_of(deg_ref)
    a = acc_ref[0] + acc_ref[1]
    h = a * dinv + b_ref[...]
    gids = lax.broadcasted_iota(jnp.int32, (1, _G), 1)
    onehot = (batch_ref[...] == gids).astype(jnp.float32)
    dn = (((0,), (0,)), ((), ()))
    pooled = lax.dot_general(onehot, h, dn,
                             preferred_element_type=jnp.float32,
                             precision=lax.Precision.DEFAULT)
    ones = jnp.ones((_NP, 1), jnp.float32)
    counts = lax.dot_general(onehot, ones, dn,
                             preferred_element_type=jnp.float32,
                             precision=lax.Precision.DEFAULT)
    o_ref[...] = pooled / jnp.maximum(counts, 1.0)


def kernel(x, edge_index, batch, W1, b1, W2, b2):
    srcw = edge_index[0].reshape(_NW, _NSEG, _CSEG, _C)
    dstw = edge_index[1].reshape(_NW, _NSEG, _CSEG, _C)
    xp = jnp.concatenate([x, jnp.zeros((_NP - _N, _D), x.dtype)])
    batch2d = jnp.concatenate(
        [batch, jnp.full((_NP - _N,), _G, batch.dtype)]).reshape(_NP, 1)
    b1r = b1.reshape(1, _D)
    b2r = b2.reshape(1, _D)

    deg = _deg(dstw).reshape(_NC, _NP, 1)
    hp1 = _tc(_mm_scale_body, (_NP, _D), deg, xp, W1)
    acc1 = _prop(hp1, srcw, dstw)
    hp2 = _tc(_layer2_body, (_NP, _D), deg, acc1, W2, b1r)
    acc2 = _prop(hp2, srcw, dstw)
    return _tc(_final_body, (_G, _D), deg, acc2, b2r, batch2d)


# speedup vs baseline: 1.0239x; 1.0201x over previous
import functools

import jax
import jax.numpy as jnp
from jax import lax
from jax.experimental import pallas as pl
from jax.experimental.pallas import tpu as pltpu
from jax.experimental.pallas import tpu_sc as plsc

_N = 10000
_E = 320000
_D = 128
_G = 64
_NP = 10240
_NC = 2
_NS = 16
_NW = _NC * _NS
_EPW = _E // _NW
_C = 80
_CPW = _EPW // _C
_NSEG = 5
_CSEG = _CPW // _NSEG
_RPT = _NP // _NS


def _fill1d(ref, n, value):
    v = jnp.full((16,), value, jnp.float32)

    def body(i, carry):
        ref[pl.ds(i * 16, 16)] = v
        return carry

    lax.fori_loop(0, n // 16, body, None)


def _sc_mesh():
    return plsc.VectorSubcoreMesh(
        core_axis_name="c", subcore_axis_name="s",
        num_cores=_NC, num_subcores=_NS)


def _deg(dstw):
    @functools.partial(
        pl.kernel,
        out_type=jax.ShapeDtypeStruct((_NC * _NP,), jnp.float32),
        mesh=_sc_mesh(),
        scratch_types=[
            pltpu.VMEM((_CSEG, _C), jnp.int32),
            pltpu.VMEM((_C,), jnp.float32),
            pltpu.VMEM((_RPT,), jnp.float32),
            pltpu.VMEM_SHARED((_NP,), jnp.float32),
            pltpu.SemaphoreType.DMA,
        ],
    )
    def run(dst_hbm, deg_out, didx, ones_v, init_v, acc_sp, sem):
        cid = lax.axis_index("c")
        sid = lax.axis_index("s")
        w = cid * _NS + sid
        _fill1d(ones_v, _C, 1.0)
        _fill1d(init_v, _RPT, 0.5)
        pltpu.sync_copy(init_v, acc_sp.at[pl.ds(sid * _RPT, _RPT)])
        plsc.subcore_barrier()

        def seg(si, carry):
            pltpu.sync_copy(dst_hbm.at[w, si], didx)

            def group(k, c2):
                descs = [
                    pltpu.async_copy(ones_v, acc_sp.at[didx.at[4 * k + j]],
                                     sem, add=True)
                    for j in range(4)
                ]
                for d in descs:
                    d.wait()
                return c2

            lax.fori_loop(0, _CSEG // 4, group, None)
            pltpu.sync_copy(ones_v, acc_sp.at[didx.at[_CSEG - 1]], add=True)
            return carry

        lax.fori_loop(0, _NSEG, seg, None)
        plsc.subcore_barrier()
        pltpu.sync_copy(acc_sp.at[pl.ds(sid * _RPT, _RPT)],
                        deg_out.at[pl.ds(cid * _NP + sid * _RPT, _RPT)])

    return run(dstw)


def _prop(hp, srcw, dstw):
    @functools.partial(
        pl.kernel,
        out_type=jax.ShapeDtypeStruct((_NC, _NP, _D), jnp.float32),
        mesh=_sc_mesh(),
        scratch_types=[
            pltpu.VMEM((_CSEG, _C), jnp.int32),
            pltpu.VMEM((_CSEG, _C), jnp.int32),
            pltpu.VMEM((_C, _D), jnp.float32),
            pltpu.VMEM((_C, _D), jnp.float32),
            pltpu.VMEM((_C, _D), jnp.float32),
            pltpu.VMEM_SHARED((_NP, _D), jnp.float32),
            [pltpu.SemaphoreType.DMA] * 3,
            [pltpu.SemaphoreType.DMA] * 3,
        ],
    )
    def run(hp_hbm, src_hbm, dst_hbm, acc_out, sidx, didx, r0, r1, r2,
            acc_sp, gsems, ssems):
        cid = lax.axis_index("c")
        sid = lax.axis_index("s")
        w = cid * _NS + sid
        row0 = sid * _RPT
        rows = (r0, r1, r2)

        @pl.when(cid == 0)
        def _():
            pltpu.sync_copy(hp_hbm.at[pl.ds(row0, _RPT)],
                            acc_sp.at[pl.ds(row0, _RPT)])

        @pl.when(cid == 1)
        def _():
            def zrow(i, carry):
                r0[i, pl.ds(0, 16)] = jnp.zeros((16,), jnp.float32)
                for jj in range(1, _D // 16):
                    r0[i, pl.ds(jj * 16, 16)] = jnp.zeros((16,), jnp.float32)
                return carry

            lax.fori_loop(0, _C, zrow, None)
            for j in range(_RPT // _C):
                pltpu.sync_copy(r0, acc_sp.at[pl.ds(row0 + j * _C, _C)])

        plsc.subcore_barrier()

        def g(c, b):
            pltpu.async_copy(hp_hbm.at[sidx.at[c]], rows[b], gsems[b])

        def wg(c, b):
            pltpu.make_async_copy(
                hp_hbm.at[sidx.at[c]], rows[b], gsems[b]).wait()

        def sc_(c, b):
            pltpu.async_copy(rows[b], acc_sp.at[didx.at[c]], ssems[b],
                             add=True)

        def ws(c, b):
            pltpu.make_async_copy(
                rows[b], acc_sp.at[didx.at[c]], ssems[b]).wait()

        def seg(si, carry):
            di = pltpu.async_copy(src_hbm.at[w, si], sidx, gsems[0])
            dj = pltpu.async_copy(dst_hbm.at[w, si], didx, gsems[1])
            di.wait()
            dj.wait()
            g(0, 0)
            g(1, 1)
            wg(0, 0)
            sc_(0, 0)
            g(2, 2)

            def trip(k, c3):
                c = 3 * k + 1
                for j, b in ((0, 1), (1, 2), (2, 0)):
                    wg(c + j, b)
                    sc_(c + j, b)
                    bp = (b + 2) % 3
                    ws(c + j - 1, bp)
                    g(c + j + 2, bp)
                return c3

            lax.fori_loop(0, (_CSEG - 4) // 3, trip, None)
            wg(_CSEG - 3, 1)
            sc_(_CSEG - 3, 1)
            ws(_CSEG - 4, 0)
            g(_CSEG - 1, 0)
            wg(_CSEG - 2, 2)
            sc_(_CSEG - 2, 2)
            wg(_CSEG - 1, 0)
            sc_(_CSEG - 1, 0)
            ws(_CSEG - 3, 1)
            ws(_CSEG - 2, 2)
            ws(_CSEG - 1, 0)
            return carry

        lax.fori_loop(0, _NSEG, seg, None)
        plsc.subcore_barrier()
        pltpu.sync_copy(acc_sp.at[pl.ds(row0, _RPT)],
                        acc_out.at[cid, pl.ds(row0, _RPT)])

    return run(hp, srcw, dstw)


def _dinv_of(deg_ref):
    d = deg_ref[0] + deg_ref[1]
    return lax.rsqrt(d)


def _mm_scale_body(deg_ref, x_ref, w_ref, o_ref):
    h = jnp.dot(x_ref[...], w_ref[...],
                preferred_element_type=jnp.float32,
                precision=lax.Precision.DEFAULT)
    o_ref[...] = h * _dinv_of(deg_ref)


def _layer2_body(deg_ref, acc_ref, w_ref, b_ref, o_ref):
    dinv = _dinv_of(deg_ref)
    a = acc_ref[0] + acc_ref[1]
    t = jnp.maximum(a * dinv + b_ref[...], 0.0)
    o_ref[...] = jnp.dot(t, w_ref[...],
                         preferred_element_type=jnp.float32,
                         precision=lax.Precision.DEFAULT) * dinv


def _tc(body, out_shape, *args):
    return pl.pallas_call(
        body, out_shape=jax.ShapeDtypeStruct(out_shape, jnp.float32))(*args)


def _final_body(deg_ref, acc_ref, b_ref, batch_ref, o_ref):
    dinv = _dinv_of(deg_ref)
    a = acc_ref[0] + acc_ref[1]
    h = a * dinv + b_ref[...]
    gids = lax.broadcasted_iota(jnp.int32, (1, _G), 1)
    onehot = (batch_ref[...] == gids).astype(jnp.float32)
    dn = (((0,), (0,)), ((), ()))
    pooled = lax.dot_general(onehot, h, dn,
                             preferred_element_type=jnp.float32,
                             precision=lax.Precision.DEFAULT)
    ones = jnp.ones((_NP, 1), jnp.float32)
    counts = lax.dot_general(onehot, ones, dn,
                             preferred_element_type=jnp.float32,
                             precision=lax.Precision.DEFAULT)
    o_ref[...] = pooled / jnp.maximum(counts, 1.0)


def kernel(x, edge_index, batch, W1, b1, W2, b2):
    srcw = edge_index[0].reshape(_NW, _NSEG, _CSEG, _C)
    dstw = edge_index[1].reshape(_NW, _NSEG, _CSEG, _C)
    xp = jnp.concatenate([x, jnp.zeros((_NP - _N, _D), x.dtype)])
    batch2d = jnp.concatenate(
        [batch, jnp.full((_NP - _N,), _G, batch.dtype)]).reshape(_NP, 1)
    b1r = b1.reshape(1, _D)
    b2r = b2.reshape(1, _D)

    deg = _deg(dstw).reshape(_NC, _NP, 1)
    hp1 = _tc(_mm_scale_body, (_NP, _D), deg, xp, W1)
    acc1 = _prop(hp1, srcw, dstw)
    hp2 = _tc(_layer2_body, (_NP, _D), deg, acc1, W2, b1r)
    acc2 = _prop(hp2, srcw, dstw)
    return _tc(_final_body, (_G, _D), deg, acc2, b2r, batch2d)


# speedup vs baseline: 1.0423x; 1.0180x over previous
import functools

import jax
import jax.numpy as jnp
from jax import lax
from jax.experimental import pallas as pl
from jax.experimental.pallas import tpu as pltpu
from jax.experimental.pallas import tpu_sc as plsc

_N = 10000
_E = 320000
_D = 128
_G = 64
_NP = 10240
_NC = 2
_NS = 16
_NW = _NC * _NS
_EPW = _E // _NW
_C = 80
_CPW = _EPW // _C
_NSEG = 5
_CSEG = _CPW // _NSEG
_RPT = _NP // _NS


def _fill1d(ref, n, value):
    v = jnp.full((16,), value, jnp.float32)

    def body(i, carry):
        ref[pl.ds(i * 16, 16)] = v
        return carry

    lax.fori_loop(0, n // 16, body, None)


def _sc_mesh():
    return plsc.VectorSubcoreMesh(
        core_axis_name="c", subcore_axis_name="s",
        num_cores=_NC, num_subcores=_NS)


def _deg(dstw):
    @functools.partial(
        pl.kernel,
        out_type=jax.ShapeDtypeStruct((_NC * _NP,), jnp.float32),
        mesh=_sc_mesh(),
        scratch_types=[
            pltpu.VMEM((_CSEG, _C), jnp.int32),
            pltpu.VMEM((_C,), jnp.float32),
            pltpu.VMEM((_RPT,), jnp.float32),
            pltpu.VMEM_SHARED((_NP,), jnp.float32),
            pltpu.SemaphoreType.DMA,
        ],
    )
    def run(dst_hbm, deg_out, didx, ones_v, init_v, acc_sp, sem):
        cid = lax.axis_index("c")
        sid = lax.axis_index("s")
        w = cid * _NS + sid
        _fill1d(ones_v, _C, 1.0)
        _fill1d(init_v, _RPT, 0.5)
        pltpu.sync_copy(init_v, acc_sp.at[pl.ds(sid * _RPT, _RPT)])
        plsc.subcore_barrier()

        def seg(si, carry):
            pltpu.sync_copy(dst_hbm.at[w, si], didx)

            def group(k, c2):
                descs = [
                    pltpu.async_copy(ones_v, acc_sp.at[didx.at[4 * k + j]],
                                     sem, add=True)
                    for j in range(4)
                ]
                for d in descs:
                    d.wait()
                return c2

            lax.fori_loop(0, _CSEG // 4, group, None)
            pltpu.sync_copy(ones_v, acc_sp.at[didx.at[_CSEG - 1]], add=True)
            return carry

        lax.fori_loop(0, _NSEG, seg, None)
        plsc.subcore_barrier()
        pltpu.sync_copy(acc_sp.at[pl.ds(sid * _RPT, _RPT)],
                        deg_out.at[pl.ds(cid * _NP + sid * _RPT, _RPT)])

    return run(dstw)


def _prop(hp, srcw, dstw):
    @functools.partial(
        pl.kernel,
        out_type=jax.ShapeDtypeStruct((_NC, _NP, _D), jnp.float32),
        mesh=_sc_mesh(),
        scratch_types=[
            pltpu.VMEM((_CSEG, _C), jnp.int32),
            pltpu.VMEM((_CSEG, _C), jnp.int32),
            pltpu.VMEM((_CSEG, _C), jnp.int32),
            pltpu.VMEM((_CSEG, _C), jnp.int32),
            pltpu.VMEM((_C, _D), jnp.float32),
            pltpu.VMEM((_C, _D), jnp.float32),
            pltpu.VMEM((_C, _D), jnp.float32),
            pltpu.VMEM_SHARED((_NP, _D), jnp.float32),
            [pltpu.SemaphoreType.DMA] * 3,
            [pltpu.SemaphoreType.DMA] * 3,
            [pltpu.SemaphoreType.DMA] * 2,
        ],
    )
    def run(hp_hbm, src_hbm, dst_hbm, acc_out, sidx0, sidx1, didx0, didx1,
            r0, r1, r2, acc_sp, gsems, ssems, isems):
        cid = lax.axis_index("c")
        sid = lax.axis_index("s")
        w = cid * _NS + sid
        row0 = sid * _RPT
        rows = (r0, r1, r2)
        sidxs = (sidx0, sidx1)
        didxs = (didx0, didx1)

        def iload(si, bp):
            pltpu.async_copy(src_hbm.at[w, si], sidxs[bp], isems[0])
            pltpu.async_copy(dst_hbm.at[w, si], didxs[bp], isems[1])

        def iwait(si, bp):
            pltpu.make_async_copy(src_hbm.at[w, si], sidxs[bp],
                                  isems[0]).wait()
            pltpu.make_async_copy(dst_hbm.at[w, si], didxs[bp],
                                  isems[1]).wait()

        iload(0, 0)

        @pl.when(cid == 0)
        def _():
            pltpu.sync_copy(hp_hbm.at[pl.ds(row0, _RPT)],
                            acc_sp.at[pl.ds(row0, _RPT)])

        @pl.when(cid == 1)
        def _():
            def zrow(i, carry):
                r0[i, pl.ds(0, 16)] = jnp.zeros((16,), jnp.float32)
                for jj in range(1, _D // 16):
                    r0[i, pl.ds(jj * 16, 16)] = jnp.zeros((16,), jnp.float32)
                return carry

            lax.fori_loop(0, _C, zrow, None)
            for j in range(_RPT // _C):
                pltpu.sync_copy(r0, acc_sp.at[pl.ds(row0 + j * _C, _C)])

        plsc.subcore_barrier()

        for si in range(_NSEG):
            bp = si % 2
            sidx = sidxs[bp]
            didx = didxs[bp]
            iwait(si, bp)
            if si + 1 < _NSEG:
                iload(si + 1, 1 - bp)

            def g(c, b):
                pltpu.async_copy(hp_hbm.at[sidx.at[c]], rows[b], gsems[b])

            def wg(c, b):
                pltpu.make_async_copy(
                    hp_hbm.at[sidx.at[c]], rows[b], gsems[b]).wait()

            def sc_(c, b):
                pltpu.async_copy(rows[b], acc_sp.at[didx.at[c]], ssems[b],
                                 add=True)

            def ws(c, b):
                pltpu.make_async_copy(
                    rows[b], acc_sp.at[didx.at[c]], ssems[b]).wait()

            g(0, 0)
            g(1, 1)
            wg(0, 0)
            sc_(0, 0)
            g(2, 2)

            def trip(k, c3):
                c = 3 * k + 1
                for j, b in ((0, 1), (1, 2), (2, 0)):
                    wg(c + j, b)
                    sc_(c + j, b)
                    bpv = (b + 2) % 3
                    ws(c + j - 1, bpv)
                    g(c + j + 2, bpv)
                return c3

            lax.fori_loop(0, (_CSEG - 4) // 3, trip, None)
            wg(_CSEG - 3, 1)
            sc_(_CSEG - 3, 1)
            ws(_CSEG - 4, 0)
            g(_CSEG - 1, 0)
            wg(_CSEG - 2, 2)
            sc_(_CSEG - 2, 2)
            wg(_CSEG - 1, 0)
            sc_(_CSEG - 1, 0)
            ws(_CSEG - 3, 1)
            ws(_CSEG - 2, 2)
            ws(_CSEG - 1, 0)

        plsc.subcore_barrier()
        pltpu.sync_copy(acc_sp.at[pl.ds(row0, _RPT)],
                        acc_out.at[cid, pl.ds(row0, _RPT)])

    return run(hp, srcw, dstw)


def _dinv_of(deg_ref):
    d = deg_ref[0] + deg_ref[1]
    return lax.rsqrt(d)


def _mm_scale_body(deg_ref, x_ref, w_ref, o_ref):
    h = jnp.dot(x_ref[...], w_ref[...],
                preferred_element_type=jnp.float32,
                precision=lax.Precision.DEFAULT)
    o_ref[...] = h * _dinv_of(deg_ref)


def _layer2_body(deg_ref, acc_ref, w_ref, b_ref, o_ref):
    dinv = _dinv_of(deg_ref)
    a = acc_ref[0] + acc_ref[1]
    t = jnp.maximum(a * dinv + b_ref[...], 0.0)
    o_ref[...] = jnp.dot(t, w_ref[...],
                         preferred_element_type=jnp.float32,
                         precision=lax.Precision.DEFAULT) * dinv


def _tc(body, out_shape, *args):
    return pl.pallas_call(
        body, out_shape=jax.ShapeDtypeStruct(out_shape, jnp.float32))(*args)


def _final_body(deg_ref, acc_ref, b_ref, batch_ref, o_ref):
    dinv = _dinv_of(deg_ref)
    a = acc_ref[0] + acc_ref[1]
    h = a * dinv + b_ref[...]
    gids = lax.broadcasted_iota(jnp.int32, (1, _G), 1)
    onehot = (batch_ref[...] == gids).astype(jnp.float32)
    dn = (((0,), (0,)), ((), ()))
    pooled = lax.dot_general(onehot, h, dn,
                             preferred_element_type=jnp.float32,
                             precision=lax.Precision.DEFAULT)
    ones = jnp.ones((_NP, 1), jnp.float32)
    counts = lax.dot_general(onehot, ones, dn,
                             preferred_element_type=jnp.float32,
                             precision=lax.Precision.DEFAULT)
    o_ref[...] = pooled / jnp.maximum(counts, 1.0)


def kernel(x, edge_index, batch, W1, b1, W2, b2):
    srcw = edge_index[0].reshape(_NW, _NSEG, _CSEG, _C)
    dstw = edge_index[1].reshape(_NW, _NSEG, _CSEG, _C)
    xp = jnp.concatenate([x, jnp.zeros((_NP - _N, _D), x.dtype)])
    batch2d = jnp.concatenate(
        [batch, jnp.full((_NP - _N,), _G, batch.dtype)]).reshape(_NP, 1)
    b1r = b1.reshape(1, _D)
    b2r = b2.reshape(1, _D)

    deg = _deg(dstw).reshape(_NC, _NP, 1)
    hp1 = _tc(_mm_scale_body, (_NP, _D), deg, xp, W1)
    acc1 = _prop(hp1, srcw, dstw)
    hp2 = _tc(_layer2_body, (_NP, _D), deg, acc1, W2, b1r)
    acc2 = _prop(hp2, srcw, dstw)
    return _tc(_final_body, (_G, _D), deg, acc2, b2r, batch2d)
